# baseline (speedup 1.0000x reference)
import functools

import jax
import jax.numpy as jnp
from jax.experimental import pallas as pl
from jax.experimental.pallas import tpu as pltpu

K = 20
LANES = 128
BIG = 1e38


def _edge_kernel(x_blk_ref, x_all_ref, out_ref, *, n, d, k):
    x = x_blk_ref[0]
    xa = x_all_ref[0]
    block = x.shape[0]
    nc = n // LANES

    inner = jnp.dot(x, xa.T, preferred_element_type=jnp.float32)
    xsq = jnp.sum(x * x, axis=1, keepdims=True)
    xasq = jnp.sum(xa * xa, axis=1, keepdims=True).T
    dist = xsq + (-2.0 * inner) + xasq

    lane = jax.lax.broadcasted_iota(
        jnp.int32, (block, LANES), 1).astype(jnp.float32)
    nf = jnp.float32(n)

    def argmin_lex(dm):
        runval = dm[:, 0:LANES]
        runchunk = jnp.zeros_like(runval)
        for c in range(1, nc):
            dc = dm[:, c * LANES:(c + 1) * LANES]
            cond = dc < runval
            runchunk = jnp.where(cond, jnp.float32(c), runchunk)
            runval = jnp.minimum(dc, runval)
        runidx = runchunk * LANES + lane
        mval = jnp.min(runval, axis=-1, keepdims=True)
        return jnp.min(jnp.where(runval == mval, runidx, nf),
                       axis=-1, keepdims=True)

    first = argmin_lex(dist)
    dm = dist
    neighbors = []
    for r in range(k):
        cstar = jnp.floor(first * (1.0 / LANES))
        lstar = first - cstar * LANES
        lanehot = (lane == lstar).astype(jnp.float32)
        ohs = []
        for c in range(nc):
            chunkhot = (cstar == jnp.float32(c)).astype(jnp.float32)
            ohs.append(lanehot * chunkhot)
        oh = jnp.concatenate(ohs, axis=1)
        neighbors.append(jnp.dot(oh, xa, preferred_element_type=jnp.float32))
        if r < k - 1:
            dm = dm + oh * BIG
            first = argmin_lex(dm)

    for j in range(k):
        base = j * 2 * d
        out_ref[0, :, base:base + d] = x
        out_ref[0, :, base + d:base + 2 * d] = neighbors[j] - x


def kernel(inputs):
    b, n, d = inputs.shape
    block = 512
    grid = (b, n // block)

    out = pl.pallas_call(
        functools.partial(_edge_kernel, n=n, d=d, k=K),
        grid=grid,
        in_specs=[
            pl.BlockSpec((1, block, d), lambda i, j: (i, j, 0)),
            pl.BlockSpec((1, n, d), lambda i, j: (i, 0, 0)),
        ],
        out_specs=pl.BlockSpec((1, block, 2 * d * K), lambda i, j: (i, j, 0)),
        out_shape=jax.ShapeDtypeStruct((b, n, 2 * d * K), jnp.float32),
    )(inputs, inputs)
    return out.reshape(b, n, K, 2 * d)
